# baseline (speedup 1.0000x reference)
import math
import functools

import jax
import jax.numpy as jnp
from jax import lax
from jax.experimental import pallas as pl
from jax.experimental.pallas import tpu as pltpu

NUM_BUCKETS = 32
MAX_DISTANCE = 128
NUM_HEADS = 16
SEQ = 4096

R = 128
LROW = 8064
EXT = 8192


def _bias_vec(table_col):
    t = lax.broadcasted_iota(jnp.int32, (1, EXT), 1)
    d = t - (SEQ - 1)
    n = -d
    ret = jnp.where(n < 0, NUM_BUCKETS // 2, 0)
    na = jnp.abs(n)
    max_exact = NUM_BUCKETS // 4
    is_small = na < max_exact
    naf = na.astype(jnp.float32)
    val = max_exact + (
        jnp.log(naf / max_exact)
        / math.log(MAX_DISTANCE / max_exact)
        * (NUM_BUCKETS // 2 - max_exact)
    ).astype(jnp.int32)
    val = jnp.minimum(val, NUM_BUCKETS // 2 - 1)
    bucket = ret + jnp.where(is_small, na, val)
    acc = jnp.zeros((1, EXT), jnp.float32)
    for b in range(NUM_BUCKETS):
        acc = jnp.where(bucket == b, table_col[b], acc)
    return acc


def _tc_body(table_ref, out_ref, bias_ref):
    h = pl.program_id(0)
    g = pl.program_id(1)

    @pl.when(g == 0)
    def _():
        vec = _bias_vec(table_ref[0, 0, :])
        for p in range(R):
            bias_ref[p, :] = vec[0, (R - 1 - p):(R - 1 - p) + LROW]

    i0 = g * R
    u0 = (SEQ - R) - i0
    out_ref[0, :, :] = bias_ref[:, pl.ds(u0, SEQ)]


def _tc_call(table3, interpret=False):
    return pl.pallas_call(
        _tc_body,
        grid=(NUM_HEADS, SEQ // R),
        in_specs=[pl.BlockSpec((1, 1, NUM_BUCKETS), lambda h, g: (h, 0, 0))],
        out_specs=pl.BlockSpec((1, R, SEQ), lambda h, g: (h, g, 0)),
        out_shape=jax.ShapeDtypeStruct((NUM_HEADS, SEQ, SEQ), jnp.float32),
        scratch_shapes=[pltpu.VMEM((R, LROW), jnp.float32)],
        interpret=interpret,
    )(table3)


from jax.experimental.pallas import tpu_sc as plsc

NSH = 8
LS = 6272
EXTS = 8448
HALF = SEQ // 2


def _stage_body(table_ref, bias_ref):
    t = lax.broadcasted_iota(jnp.int32, (1, EXTS), 1)
    d = t - (SEQ - 1)
    n = -d
    ret = jnp.where(n < 0, NUM_BUCKETS // 2, 0)
    na = jnp.abs(n)
    max_exact = NUM_BUCKETS // 4
    is_small = na < max_exact
    naf = na.astype(jnp.float32)
    val = max_exact + (
        jnp.log(naf / max_exact)
        / math.log(MAX_DISTANCE / max_exact)
        * (NUM_BUCKETS // 2 - max_exact)
    ).astype(jnp.int32)
    val = jnp.minimum(val, NUM_BUCKETS // 2 - 1)
    bucket = ret + jnp.where(is_small, na, val)
    acc = jnp.zeros((1, EXTS), jnp.float32)
    for b in range(NUM_BUCKETS):
        acc = jnp.where(bucket == b, table_ref[0, 0, b], acc)
    for half in range(2):
        for r in range(NSH):
            base = (1 - half) * HALF + r
            bias_ref[0, half, r, :] = acc[0, base:base + LS]


def _stage_call(table3):
    return pl.pallas_call(
        _stage_body,
        grid=(NUM_HEADS,),
        in_specs=[pl.BlockSpec((1, 1, NUM_BUCKETS), lambda h: (h, 0, 0))],
        out_specs=pl.BlockSpec((1, 2, NSH, LS), lambda h: (h, 0, 0, 0)),
        out_shape=jax.ShapeDtypeStruct((NUM_HEADS, 2, NSH, LS), jnp.float32),
    )(table3)


_SC_MESH = plsc.VectorSubcoreMesh(core_axis_name="c", subcore_axis_name="s")
_SC_DEPTH = 8


@functools.partial(
    pl.kernel,
    mesh=_SC_MESH,
    out_type=jax.ShapeDtypeStruct((NUM_HEADS * SEQ * SEQ,), jnp.float32),
    scratch_types=[
        pltpu.VMEM((NSH * LS,), jnp.float32),
        pltpu.SemaphoreType.DMA,
    ],
)
def _sc_write(bias_hbm, out_hbm, bias_v, sem):
    c = lax.axis_index("c")
    s = lax.axis_index("s")
    wid = s * 2 + c
    h = wid // 2
    half = wid % 2
    for r in range(NSH):
        pltpu.sync_copy(bias_hbm.at[h, half, r], bias_v.at[pl.ds(r * LS, LS)])
    i_base = half * HALF

    unroll = 4

    def body(t, carry):
        for b in range(unroll):
            li = t * unroll + b
            s_loc = (HALF - 1) - li
            r = jnp.bitwise_and(s_loc, NSH - 1)
            off = pl.multiple_of(r * LS + (s_loc - r), 8)
            dst_off = pl.multiple_of((h * SEQ + i_base + li) * SEQ, SEQ)
            pltpu.async_copy(
                bias_v.at[pl.ds(off, SEQ)],
                out_hbm.at[pl.ds(dst_off, SEQ)],
                sem,
            )

        @pl.when(t >= _SC_DEPTH // unroll)
        def _():
            for _ in range(unroll):
                pltpu.make_async_copy(
                    bias_v.at[pl.ds(0, SEQ)],
                    out_hbm.at[pl.ds(0, SEQ)],
                    sem,
                ).wait()

        return carry

    lax.fori_loop(0, HALF // unroll, body, 0)
    for _ in range(_SC_DEPTH):
        pltpu.make_async_copy(
            bias_v.at[pl.ds(0, SEQ)],
            out_hbm.at[pl.ds(0, SEQ)],
            sem,
        ).wait()


def kernel(query_length, key_length, rel_bias_table):
    del query_length, key_length
    table3 = rel_bias_table.T.reshape(NUM_HEADS, 1, NUM_BUCKETS)
    bias = _stage_call(table3)
    out = _sc_write(bias)
    return out.reshape(1, NUM_HEADS, SEQ, SEQ)


# speedup vs baseline: 3.0874x; 3.0874x over previous
import math
import functools

import jax
import jax.numpy as jnp
from jax import lax
from jax.experimental import pallas as pl
from jax.experimental.pallas import tpu as pltpu

NUM_BUCKETS = 32
MAX_DISTANCE = 128
NUM_HEADS = 16
SEQ = 4096

R = 128
LROW = 8064
EXT = 8192


def _bias_vec(table_col):
    t = lax.broadcasted_iota(jnp.int32, (1, EXT), 1)
    d = t - (SEQ - 1)
    n = -d
    ret = jnp.where(n < 0, NUM_BUCKETS // 2, 0)
    na = jnp.abs(n)
    max_exact = NUM_BUCKETS // 4
    is_small = na < max_exact
    naf = na.astype(jnp.float32)
    val = max_exact + (
        jnp.log(naf / max_exact)
        / math.log(MAX_DISTANCE / max_exact)
        * (NUM_BUCKETS // 2 - max_exact)
    ).astype(jnp.int32)
    val = jnp.minimum(val, NUM_BUCKETS // 2 - 1)
    bucket = ret + jnp.where(is_small, na, val)
    acc = jnp.zeros((1, EXT), jnp.float32)
    for b in range(NUM_BUCKETS):
        acc = jnp.where(bucket == b, table_col[b], acc)
    return acc


def _tc_body(table_ref, out_ref, bias_ref):
    h = pl.program_id(0)
    g = pl.program_id(1)

    @pl.when(g == 0)
    def _():
        vec = _bias_vec(table_ref[0, 0, :])
        for p in range(R):
            bias_ref[p, :] = vec[0, (R - 1 - p):(R - 1 - p) + LROW]

    i0 = g * R
    u0 = (SEQ - R) - i0
    out_ref[0, :, :] = bias_ref[:, pl.ds(u0, SEQ)]


def _tc_call(table3, interpret=False):
    return pl.pallas_call(
        _tc_body,
        grid=(NUM_HEADS, SEQ // R),
        in_specs=[pl.BlockSpec((1, 1, NUM_BUCKETS), lambda h, g: (h, 0, 0))],
        out_specs=pl.BlockSpec((1, R, SEQ), lambda h, g: (h, g, 0)),
        out_shape=jax.ShapeDtypeStruct((NUM_HEADS, SEQ, SEQ), jnp.float32),
        scratch_shapes=[pltpu.VMEM((R, LROW), jnp.float32)],
        interpret=interpret,
    )(table3)


from jax.experimental.pallas import tpu_sc as plsc

MCLS = 8
KPG = 16
NG = 16
LM = 6016
UPT = 8
EXTS = 8192
NBLK = NUM_HEADS * SEQ // KPG


def _stage_body(table_ref, bias_ref):
    acc = _bias_vec_sc(table_ref)
    for half in range(2):
        for m in range(MCLS):
            off0 = 2160 - 2048 * half - 16 * m
            for p in range(KPG):
                base = (KPG - 1 - p) + off0
                bias_ref[0, half, m, p, :] = acc[0, base:base + LM]


def _bias_vec_sc(table_ref):
    t = lax.broadcasted_iota(jnp.int32, (1, EXTS), 1)
    d = t - (SEQ - 1)
    n = -d
    ret = jnp.where(n < 0, NUM_BUCKETS // 2, 0)
    na = jnp.abs(n)
    max_exact = NUM_BUCKETS // 4
    is_small = na < max_exact
    naf = na.astype(jnp.float32)
    val = max_exact + (
        jnp.log(naf / max_exact)
        / math.log(MAX_DISTANCE / max_exact)
        * (NUM_BUCKETS // 2 - max_exact)
    ).astype(jnp.int32)
    val = jnp.minimum(val, NUM_BUCKETS // 2 - 1)
    bucket = ret + jnp.where(is_small, na, val)
    acc = jnp.zeros((1, EXTS), jnp.float32)
    for b in range(NUM_BUCKETS):
        acc = jnp.where(bucket == b, table_ref[0, 0, b], acc)
    return acc


def _stage_call(table3):
    return pl.pallas_call(
        _stage_body,
        grid=(NUM_HEADS,),
        in_specs=[pl.BlockSpec((1, 1, NUM_BUCKETS), lambda h: (h, 0, 0))],
        out_specs=pl.BlockSpec((1, 2, MCLS, KPG, LM), lambda h: (h, 0, 0, 0, 0)),
        out_shape=jax.ShapeDtypeStruct((NUM_HEADS, 2, MCLS, KPG, LM), jnp.float32),
    )(table3)


_SC_MESH = plsc.VectorSubcoreMesh(core_axis_name="c", subcore_axis_name="s")


@functools.partial(
    pl.kernel,
    mesh=_SC_MESH,
    out_type=jax.ShapeDtypeStruct((NBLK, KPG, SEQ), jnp.float32),
    scratch_types=[
        pltpu.VMEM((KPG, LM), jnp.float32),
        pltpu.SemaphoreType.DMA,
    ],
)
def _sc_write(bias_hbm, out_hbm, bias_v, sem):
    c = lax.axis_index("c")
    s = lax.axis_index("s")
    wid = s * 2 + c

    def drain_one(k, carry):
        pltpu.make_async_copy(
            bias_v.at[:, pl.ds(0, SEQ)],
            out_hbm.at[0],
            sem,
        ).wait()
        return carry

    def unit(j, carry):
        u = wid * UPT + j
        h = u // (2 * MCLS)
        rem = lax.rem(u, 2 * MCLS)
        half = rem // MCLS
        m = lax.rem(rem, MCLS)

        @pl.when(j > 0)
        def _():
            lax.fori_loop(0, NG, drain_one, 0)

        pltpu.sync_copy(bias_hbm.at[h, half, m], bias_v)

        def store(k, cc):
            w0 = pl.multiple_of(128 * (NG - 1) - 128 * k, 128)
            g = 128 * half + MCLS * k + m
            pltpu.async_copy(
                bias_v.at[:, pl.ds(w0, SEQ)],
                out_hbm.at[h * (SEQ // KPG) + g],
                sem,
            )
            return cc

        lax.fori_loop(0, NG, store, 0)
        return carry

    lax.fori_loop(0, UPT, unit, 0)
    lax.fori_loop(0, NG, drain_one, 0)


def kernel(query_length, key_length, rel_bias_table):
    del query_length, key_length
    table3 = rel_bias_table.T.reshape(NUM_HEADS, 1, NUM_BUCKETS)
    bias = _stage_call(table3)
    out = _sc_write(bias)
    return out.reshape(1, NUM_HEADS, SEQ, SEQ)
